# baseline (speedup 1.0000x reference)
import jax
import jax.numpy as jnp
import numpy as np
from jax.experimental import pallas as pl

_ZDIM = 512
_LAYERS = 8
_NUM_WS = 14
_WGAIN = 0.01 / np.sqrt(512.0)
_BGAIN = 0.01


def _mlp_kernel(z_ref, w_ref, b_ref, o_ref):
    x = z_ref[...]
    x = x * jax.lax.rsqrt(jnp.mean(x * x, axis=1, keepdims=True) + 1e-8)
    for i in range(_LAYERS):
        y = jax.lax.dot_general(
            x * _WGAIN, w_ref[i],
            (((1,), (1,)), ((), ())),
            preferred_element_type=jnp.float32,
            precision=jax.lax.Precision.HIGHEST,
        )
        y = y + b_ref[i][None, :] * _BGAIN
        x = jnp.where(y >= 0, y, 0.01 * y)
    o_ref[...] = jnp.broadcast_to(x[:, None, :], (x.shape[0], _NUM_WS, _ZDIM))


def kernel(z, c, W, b):
    del c
    batch = z.shape[0]
    m = 256
    out = pl.pallas_call(
        _mlp_kernel,
        grid=(batch // m,),
        in_specs=[
            pl.BlockSpec((m, _ZDIM), lambda i: (i, 0)),
            pl.BlockSpec((_LAYERS, _ZDIM, _ZDIM), lambda i: (0, 0, 0)),
            pl.BlockSpec((_LAYERS, _ZDIM), lambda i: (0, 0)),
        ],
        out_specs=pl.BlockSpec((m, _NUM_WS, _ZDIM), lambda i: (i, 0, 0)),
        out_shape=jax.ShapeDtypeStruct((batch, _NUM_WS, _ZDIM), jnp.float32),
    )(z, W, b)
    return out


# speedup vs baseline: 1.3010x; 1.3010x over previous
import jax
import jax.numpy as jnp
import numpy as np
from jax.experimental import pallas as pl
from jax.experimental.pallas import tpu as pltpu

_ZDIM = 512
_LAYERS = 8
_NUM_WS = 14
_WGAIN = 0.01 / np.sqrt(512.0)
_BGAIN = 0.01


def _mlp_kernel(z_ref, w_ref, b_ref, o_ref, wh_ref, wl_ref):
    @pl.when(pl.program_id(0) == 0)
    def _():
        w = w_ref[...] * _WGAIN
        wh = w.astype(jnp.bfloat16)
        wh_ref[...] = wh
        wl_ref[...] = (w - wh.astype(jnp.float32)).astype(jnp.bfloat16)

    x = z_ref[...]
    x = x * jax.lax.rsqrt(jnp.mean(x * x, axis=1, keepdims=True) + 1e-8)
    dims = (((1,), (1,)), ((), ()))
    for i in range(_LAYERS):
        xh = x.astype(jnp.bfloat16)
        xl = (x - xh.astype(jnp.float32)).astype(jnp.bfloat16)
        y = jax.lax.dot_general(xh, wl_ref[i], dims,
                                preferred_element_type=jnp.float32)
        y = y + jax.lax.dot_general(xl, wh_ref[i], dims,
                                    preferred_element_type=jnp.float32)
        y = y + jax.lax.dot_general(xh, wh_ref[i], dims,
                                    preferred_element_type=jnp.float32)
        y = y + b_ref[i][None, :] * _BGAIN
        x = jnp.where(y >= 0, y, 0.01 * y)
    o_ref[...] = jnp.broadcast_to(x[:, None, :], (x.shape[0], _NUM_WS, _ZDIM))


def kernel(z, c, W, b):
    del c
    batch = z.shape[0]
    m = 256
    out = pl.pallas_call(
        _mlp_kernel,
        grid=(batch // m,),
        in_specs=[
            pl.BlockSpec((m, _ZDIM), lambda i: (i, 0)),
            pl.BlockSpec((_LAYERS, _ZDIM, _ZDIM), lambda i: (0, 0, 0)),
            pl.BlockSpec((_LAYERS, _ZDIM), lambda i: (0, 0)),
        ],
        out_specs=pl.BlockSpec((m, _NUM_WS, _ZDIM), lambda i: (i, 0, 0)),
        out_shape=jax.ShapeDtypeStruct((batch, _NUM_WS, _ZDIM), jnp.float32),
        scratch_shapes=[
            pltpu.VMEM((_LAYERS, _ZDIM, _ZDIM), jnp.bfloat16),
            pltpu.VMEM((_LAYERS, _ZDIM, _ZDIM), jnp.bfloat16),
        ],
    )(z, W, b)
    return out
